# baseline (speedup 1.0000x reference)
import jax
import jax.numpy as jnp
from jax import lax
from jax.experimental import pallas as pl
from jax.experimental.pallas import tpu as pltpu
from jax.experimental.pallas import tpu_sc as plsc

NC, NS, L = 1, 16, 16
NW = NC * NS
B, NSENT, V, D = 4, 128, 4096, 2048
TOTAL = B * NSENT
RPW = TOTAL // NW
NCHUNK = 4
CROWS = RPW // NCHUNK


def _pool_body(table_hbm, idx_hbm, mask_hbm, out_hbm, mask_v,
               idx_c, rows_c, isems, gsems, msem, ssem):
    wid = lax.axis_index("s") * NC + lax.axis_index("c")
    base = wid * RPW
    batch = base // NSENT
    idx_copies = [
        pltpu.async_copy(idx_hbm.at[wid * NCHUNK + c], idx_c[c], isems[c])
        for c in range(NCHUNK)
    ]
    mask_copy = pltpu.async_copy(mask_hbm.at[pl.ds(base, RPW)], mask_v, msem)

    gathers = []
    for c in range(NCHUNK):
        idx_copies[c].wait()
        gathers.append(
            pltpu.async_copy(
                table_hbm.at[batch].at[idx_c[c]], rows_c[c], gsems[c]
            )
        )

    mask_copy.wait()
    m_parts = [mask_v[pl.ds(k * L, L)] for k in range(RPW // L)]
    allset = m_parts[0][0]
    for k in range(RPW // L):
        for i in range(1 if k == 0 else 0, L):
            allset = allset & m_parts[k][i]

    scatters = []
    for c in range(NCHUNK):
        gathers[c].wait()

        @pl.when(allset == 0)
        def _mask_slow_path(c=c):
            for i in range(CROWS):
                r = c * CROWS + i
                bcf = m_parts[r // L][r % L].astype(jnp.float32)

                def body(j, carry):
                    off = pl.multiple_of(j * L, L)
                    rows_c[c][i, pl.ds(off, L)] = (
                        rows_c[c][i, pl.ds(off, L)] * bcf
                    )
                    return carry

                lax.fori_loop(0, D // L, body, 0)

        scatters.append(
            pltpu.async_copy(
                rows_c[c], out_hbm.at[pl.ds(base + c * CROWS, CROWS)], ssem
            )
        )
    for c in range(NCHUNK):
        scatters[c].wait()


_mesh = plsc.VectorSubcoreMesh(
    core_axis_name="c", subcore_axis_name="s", num_cores=NC, num_subcores=NS
)

_pool = pl.kernel(
    _pool_body,
    out_type=jax.ShapeDtypeStruct((TOTAL, D), jnp.float32),
    mesh=_mesh,
    scratch_types=[
        pltpu.VMEM((RPW,), jnp.int32),
        [pltpu.VMEM((CROWS,), jnp.int32)] * NCHUNK,
        [pltpu.VMEM((CROWS, D), jnp.float32)] * NCHUNK,
        [pltpu.SemaphoreType.DMA] * NCHUNK,
        [pltpu.SemaphoreType.DMA] * NCHUNK,
        pltpu.SemaphoreType.DMA,
        pltpu.SemaphoreType.DMA,
    ],
)


def kernel(word_vectors, sent_rep_token_ids, sent_rep_mask):
    ids = sent_rep_token_ids.astype(jnp.int32).reshape(TOTAL // CROWS, CROWS)
    mask_i = sent_rep_mask.astype(jnp.int32).reshape(TOTAL)
    out = _pool(word_vectors, ids, mask_i)
    return out.reshape(B, NSENT, D), sent_rep_mask


# speedup vs baseline: 1.0688x; 1.0688x over previous
import jax
import jax.numpy as jnp
from jax import lax
from jax.experimental import pallas as pl
from jax.experimental.pallas import tpu as pltpu
from jax.experimental.pallas import tpu_sc as plsc

NC, NS, L = 2, 16, 16
NW = NC * NS
B, NSENT, V, D = 4, 128, 4096, 2048
TOTAL = B * NSENT
RPW = TOTAL // NW
NCHUNK = 2
CROWS = RPW // NCHUNK


def _pool_body(table_hbm, idx_hbm, mask_hbm, out_hbm, mask_v,
               idx_c, rows_c, isems, gsems, msem, ssem):
    wid = lax.axis_index("s") * NC + lax.axis_index("c")
    base = wid * RPW
    batch = base // NSENT
    idx_copies = [
        pltpu.async_copy(
            idx_hbm.at[pl.ds(base + c * CROWS, CROWS)], idx_c[c], isems[c]
        )
        for c in range(NCHUNK)
    ]
    mask_copy = pltpu.async_copy(mask_hbm.at[pl.ds(base, RPW)], mask_v, msem)

    gathers = []
    for c in range(NCHUNK):
        idx_copies[c].wait()
        gathers.append(
            pltpu.async_copy(
                table_hbm.at[batch].at[idx_c[c]], rows_c[c], gsems[c]
            )
        )

    mask_copy.wait()
    m = mask_v[...]
    allset = m[0]
    for i in range(1, RPW):
        allset = allset & m[i]

    scatters = []
    for c in range(NCHUNK):
        gathers[c].wait()

        @pl.when(allset == 0)
        def _mask_slow_path(c=c):
            for i in range(CROWS):
                bcf = m[c * CROWS + i].astype(jnp.float32)

                def body(j, carry):
                    off = pl.multiple_of(j * L, L)
                    rows_c[c][i, pl.ds(off, L)] = (
                        rows_c[c][i, pl.ds(off, L)] * bcf
                    )
                    return carry

                lax.fori_loop(0, D // L, body, 0)

        scatters.append(
            pltpu.async_copy(
                rows_c[c], out_hbm.at[pl.ds(base + c * CROWS, CROWS)], ssem
            )
        )
    for c in range(NCHUNK):
        scatters[c].wait()


_mesh = plsc.VectorSubcoreMesh(
    core_axis_name="c", subcore_axis_name="s", num_cores=NC, num_subcores=NS
)

_pool = pl.kernel(
    _pool_body,
    out_type=jax.ShapeDtypeStruct((TOTAL, D), jnp.float32),
    mesh=_mesh,
    scratch_types=[
        pltpu.VMEM((RPW,), jnp.int32),
        [pltpu.VMEM((CROWS,), jnp.int32)] * NCHUNK,
        [pltpu.VMEM((CROWS, D), jnp.float32)] * NCHUNK,
        [pltpu.SemaphoreType.DMA] * NCHUNK,
        [pltpu.SemaphoreType.DMA] * NCHUNK,
        pltpu.SemaphoreType.DMA,
        pltpu.SemaphoreType.DMA,
    ],
)


def kernel(word_vectors, sent_rep_token_ids, sent_rep_mask):
    ids = sent_rep_token_ids.astype(jnp.int32).reshape(TOTAL)
    mask_i = sent_rep_mask.astype(jnp.int32).reshape(TOTAL)
    out = _pool(word_vectors, ids, mask_i)
    return out.reshape(B, NSENT, D), sent_rep_mask


# speedup vs baseline: 1.0694x; 1.0006x over previous
import jax
import jax.numpy as jnp
from jax import lax
from jax.experimental import pallas as pl
from jax.experimental.pallas import tpu as pltpu
from jax.experimental.pallas import tpu_sc as plsc

NC, NS, L = 2, 16, 16
NW = NC * NS
B, NSENT, V, D = 4, 128, 4096, 2048
TOTAL = B * NSENT
RPW = TOTAL // NW
NCHUNK = 2
CROWS = RPW // NCHUNK


def _pool_body(table_hbm, idx_hbm, mask_hbm, out_hbm, mask_v,
               idx_c, rows_c, isems, gsems, msem, ssem):
    wid = lax.axis_index("s") * NC + lax.axis_index("c")
    base = wid * RPW
    batch = base // NSENT
    idx_copies = [
        pltpu.async_copy(
            idx_hbm.at[pl.ds(base + c * CROWS, CROWS)], idx_c[c], isems[c]
        )
        for c in range(NCHUNK)
    ]
    mask_copy = pltpu.async_copy(mask_hbm.at[pl.ds(base, RPW)], mask_v, msem)

    gathers = []
    for c in range(NCHUNK):
        idx_copies[c].wait()
        gathers.append(
            pltpu.async_copy(
                table_hbm.at[batch].at[idx_c[c]], rows_c[c], gsems[c]
            )
        )

    mask_copy.wait()
    m = mask_v[...]
    allset = m[0]
    for i in range(1, RPW):
        allset = allset & m[i]

    scatters = []
    for c in range(NCHUNK):
        gathers[c].wait()

        @pl.when(allset == 0)
        def _mask_slow_path(c=c):
            for i in range(CROWS):
                bcf = m[c * CROWS + i].astype(jnp.float32)

                def body(j, carry):
                    off = pl.multiple_of(j * L, L)
                    rows_c[c][i, pl.ds(off, L)] = (
                        rows_c[c][i, pl.ds(off, L)] * bcf
                    )
                    return carry

                lax.fori_loop(0, D // L, body, 0)

        scatters.append(
            pltpu.async_copy(
                rows_c[c], out_hbm.at[pl.ds(base + c * CROWS, CROWS)], ssem
            )
        )
    for c in range(NCHUNK):
        scatters[c].wait()


_mesh = plsc.VectorSubcoreMesh(
    core_axis_name="c", subcore_axis_name="s", num_cores=NC, num_subcores=NS
)

_pool = pl.kernel(
    _pool_body,
    out_type=jax.ShapeDtypeStruct((TOTAL, D), jnp.float32),
    mesh=_mesh,
    compiler_params=pltpu.CompilerParams(skip_device_barrier=True),
    scratch_types=[
        pltpu.VMEM((RPW,), jnp.int32),
        [pltpu.VMEM((CROWS,), jnp.int32)] * NCHUNK,
        [pltpu.VMEM((CROWS, D), jnp.float32)] * NCHUNK,
        [pltpu.SemaphoreType.DMA] * NCHUNK,
        [pltpu.SemaphoreType.DMA] * NCHUNK,
        pltpu.SemaphoreType.DMA,
        pltpu.SemaphoreType.DMA,
    ],
)


def kernel(word_vectors, sent_rep_token_ids, sent_rep_mask):
    ids = sent_rep_token_ids.astype(jnp.int32).reshape(TOTAL)
    mask_i = sent_rep_mask.astype(jnp.int32).reshape(TOTAL)
    out = _pool(word_vectors, ids, mask_i)
    return out.reshape(B, NSENT, D), sent_rep_mask
